# baseline (speedup 1.0000x reference)
import functools

import jax
import jax.numpy as jnp
from jax.experimental import pallas as pl
from jax.experimental.pallas import tpu as pltpu

QUARTER = 192
SEQ = 81
HID = 4 * QUARTER
BS = 9


def _one_hot_rows(idx, n, table):
    j = jax.lax.broadcasted_iota(jnp.int32, (BS, n), 1)
    oh = (j == idx[:, None]).astype(jnp.float32)
    return jax.lax.dot_general(
        oh, table, (((1,), (0,)), ((), ())),
        preferred_element_type=jnp.float32)


def _bcast_kernel(batch, row_ref, col_ref, box_ref, pos_ref, out_ref):
    i = pl.program_id(0)
    p = i * BS + jax.lax.broadcasted_iota(jnp.int32, (BS,), 0)
    r, c = p // 9, p % 9
    b = (r // 3) * 3 + c // 3
    rows = jnp.concatenate([
        _one_hot_rows(r, 9, row_ref[:]),
        _one_hot_rows(c, 9, col_ref[:]),
        _one_hot_rows(b, 9, box_ref[:]),
        _one_hot_rows(p, SEQ, pos_ref[:]),
    ], axis=-1)
    out_ref[:] = jnp.broadcast_to(rows[:, None, :], (BS, batch // 2, HID))


@functools.partial(jax.jit, static_argnames=("batch",))
def _run(row_embed, col_embed, box_embed, pos_embed, batch):
    grid = (SEQ // BS,)
    out = pl.pallas_call(
        functools.partial(_bcast_kernel, batch),
        grid=(SEQ // BS, 2),
        in_specs=[
            pl.BlockSpec((9, QUARTER), lambda i, j: (0, 0)),
            pl.BlockSpec((9, QUARTER), lambda i, j: (0, 0)),
            pl.BlockSpec((9, QUARTER), lambda i, j: (0, 0)),
            pl.BlockSpec((SEQ, QUARTER), lambda i, j: (0, 0)),
        ],
        out_specs=pl.BlockSpec((BS, batch // 2, HID), lambda i, j: (i, j, 0)),
        out_shape=jax.ShapeDtypeStruct((SEQ, batch, HID), jnp.float32),
        compiler_params=pltpu.CompilerParams(
            dimension_semantics=("parallel", "parallel"),
        ),
    )(row_embed, col_embed, box_embed, pos_embed)
    return jnp.transpose(out, (1, 0, 2))


def kernel(x, row_embed, col_embed, box_embed, pos_embed):
    batch = x.shape[0]
    return _run(row_embed, col_embed, box_embed, pos_embed, batch)


# speedup vs baseline: 1.0184x; 1.0184x over previous
import functools

import jax
import jax.numpy as jnp
from jax.experimental import pallas as pl
from jax.experimental.pallas import tpu as pltpu

QUARTER = 192
SEQ = 81
HID = 4 * QUARTER
BS = 3


def _one_hot_rows(idx, n, table):
    j = jax.lax.broadcasted_iota(jnp.int32, (BS, n), 1)
    oh = (j == idx[:, None]).astype(jnp.float32)
    return jax.lax.dot_general(
        oh, table, (((1,), (0,)), ((), ())),
        preferred_element_type=jnp.float32)


def _bcast_kernel(batch, row_ref, col_ref, box_ref, pos_ref, out_ref):
    i = pl.program_id(0)
    p = i * BS + jax.lax.broadcasted_iota(jnp.int32, (BS,), 0)
    r, c = p // 9, p % 9
    b = (r // 3) * 3 + c // 3
    rows = jnp.concatenate([
        _one_hot_rows(r, 9, row_ref[:]),
        _one_hot_rows(c, 9, col_ref[:]),
        _one_hot_rows(b, 9, box_ref[:]),
        _one_hot_rows(p, SEQ, pos_ref[:]),
    ], axis=-1)
    out_ref[:] = jnp.broadcast_to(rows[:, None, :], (BS, batch // 2, HID))


@functools.partial(jax.jit, static_argnames=("batch",))
def _run(row_embed, col_embed, box_embed, pos_embed, batch):
    grid = (SEQ // BS,)
    out = pl.pallas_call(
        functools.partial(_bcast_kernel, batch),
        grid=(SEQ // BS, 2),
        in_specs=[
            pl.BlockSpec((9, QUARTER), lambda i, j: (0, 0)),
            pl.BlockSpec((9, QUARTER), lambda i, j: (0, 0)),
            pl.BlockSpec((9, QUARTER), lambda i, j: (0, 0)),
            pl.BlockSpec((SEQ, QUARTER), lambda i, j: (0, 0)),
        ],
        out_specs=pl.BlockSpec((BS, batch // 2, HID), lambda i, j: (i, j, 0)),
        out_shape=jax.ShapeDtypeStruct((SEQ, batch, HID), jnp.float32),
        compiler_params=pltpu.CompilerParams(
            dimension_semantics=("parallel", "parallel"),
        ),
    )(row_embed, col_embed, box_embed, pos_embed)
    return jnp.transpose(out, (1, 0, 2))


def kernel(x, row_embed, col_embed, box_embed, pos_embed):
    batch = x.shape[0]
    return _run(row_embed, col_embed, box_embed, pos_embed, batch)


# speedup vs baseline: 1.0191x; 1.0007x over previous
import functools

import jax
import jax.numpy as jnp
from jax.experimental import pallas as pl
from jax.experimental.pallas import tpu as pltpu

QUARTER = 192
SEQ = 81
HID = 4 * QUARTER
BS = 3


def _one_hot_rows(idx, n, table):
    j = jax.lax.broadcasted_iota(jnp.int32, (BS, n), 1)
    oh = (j == idx[:, None]).astype(jnp.float32)
    return jax.lax.dot_general(
        oh, table, (((1,), (0,)), ((), ())),
        preferred_element_type=jnp.float32)


def _bcast_kernel(batch, row_ref, col_ref, box_ref, pos_ref, out_ref):
    i = pl.program_id(0)
    p = i * BS + jax.lax.broadcasted_iota(jnp.int32, (BS,), 0)
    r, c = p // 9, p % 9
    b = (r // 3) * 3 + c // 3
    rows = jnp.concatenate([
        _one_hot_rows(r, 9, row_ref[:]),
        _one_hot_rows(c, 9, col_ref[:]),
        _one_hot_rows(b, 9, box_ref[:]),
        _one_hot_rows(p, SEQ, pos_ref[:]),
    ], axis=-1)
    out_ref[:] = jnp.broadcast_to(rows[:, None, :], (BS, batch // 2, HID))


@functools.partial(jax.jit, static_argnames=("batch",))
def _run(row_embed, col_embed, box_embed, pos_embed, batch):
    out = pl.pallas_call(
        functools.partial(_bcast_kernel, batch),
        grid=(SEQ // BS, 2),
        in_specs=[
            pl.BlockSpec((9, QUARTER), lambda i, j: (0, 0)),
            pl.BlockSpec((9, QUARTER), lambda i, j: (0, 0)),
            pl.BlockSpec((9, QUARTER), lambda i, j: (0, 0)),
            pl.BlockSpec((SEQ, QUARTER), lambda i, j: (0, 0)),
        ],
        out_specs=pl.BlockSpec((BS, batch // 2, HID), lambda i, j: (i, j, 0)),
        out_shape=jax.ShapeDtypeStruct((SEQ, batch, HID), jnp.float32),
        compiler_params=pltpu.CompilerParams(
            dimension_semantics=("parallel", "parallel"),
        ),
    )(row_embed, col_embed, box_embed, pos_embed)
    return jnp.transpose(out, (1, 0, 2))


def kernel(x, row_embed, col_embed, box_embed, pos_embed):
    batch = x.shape[0]
    return _run(row_embed, col_embed, box_embed, pos_embed, batch)
